# baseline (speedup 1.0000x reference)
import functools
import math

import jax
import jax.numpy as jnp
from jax import lax
from jax.experimental import pallas as pl
from jax.experimental.pallas import tpu as pltpu
from jax.experimental.pallas import tpu_sc as plsc

D_MODEL = 64
SCALE = math.sqrt(D_MODEL)

NC = 2
NS = 16
NW = NC * NS

CHUNK = 128
B_TOTAL = 4096 * 200
B_PER_W = B_TOTAL // NW
CHUNKS_PER_W = B_PER_W // CHUNK


def _emb_kernel(x_hbm, table_hbm, out_hbm,
                idx_v, g0, g1, s0, s1, gsem0, gsem1, ssem0, ssem1):
    wid = lax.axis_index("s") * NC + lax.axis_index("c")
    base = wid * B_PER_W
    pltpu.sync_copy(x_hbm.at[wid], idx_v)

    gbufs, sbufs = (g0, g1), (s0, s1)
    gsems, ssems = (gsem0, gsem1), (ssem0, ssem1)

    def start_gather(g, b):
        pltpu.async_copy(table_hbm.at[idx_v.at[g]], gbufs[b], gsems[b])

    def wait_gather(b):
        pltpu.make_async_copy(table_hbm.at[idx_v.at[0]], gbufs[b],
                              gsems[b]).wait()

    def scale(b):
        def srow(r, c2):
            for c in range(D_MODEL // 16):
                sbufs[b][r, pl.ds(c * 16, 16)] = (
                    gbufs[b][r, pl.ds(c * 16, 16)] * SCALE
                )
            return c2
        lax.fori_loop(0, CHUNK, srow, 0, unroll=4)

    def start_store(g, b):
        pltpu.async_copy(sbufs[b], out_hbm.at[pl.ds(base + g * CHUNK, CHUNK)],
                         ssems[b])

    def wait_store(b):
        pltpu.make_async_copy(sbufs[b], out_hbm.at[pl.ds(base, CHUNK)],
                              ssems[b]).wait()

    start_gather(0, 0)
    start_gather(1, 1)
    for b in range(2):
        wait_gather(b)
        scale(b)
        start_store(b, b)
        start_gather(b + 2, b)

    def body(i, c):
        for b in range(2):
            g = 2 * i + b
            wait_gather(b)
            wait_store(b)
            scale(b)
            start_store(g, b)

            @pl.when(g + 2 < CHUNKS_PER_W)
            def _():
                start_gather(g + 2, b)
        return c

    lax.fori_loop(1, CHUNKS_PER_W // 2, body, 0)
    wait_store(0)
    wait_store(1)


@jax.jit
def _embedding(x_flat, table):
    mesh = plsc.VectorSubcoreMesh(core_axis_name="c", subcore_axis_name="s")
    kfn = functools.partial(
        pl.kernel,
        mesh=mesh,
        out_type=jax.ShapeDtypeStruct((B_TOTAL, D_MODEL), jnp.float32),
        scratch_types=[
            pltpu.VMEM((CHUNKS_PER_W, CHUNK), jnp.int32),
            pltpu.VMEM((CHUNK, D_MODEL), jnp.float32),
            pltpu.VMEM((CHUNK, D_MODEL), jnp.float32),
            pltpu.VMEM((CHUNK, D_MODEL), jnp.float32),
            pltpu.VMEM((CHUNK, D_MODEL), jnp.float32),
            pltpu.SemaphoreType.DMA,
            pltpu.SemaphoreType.DMA,
            pltpu.SemaphoreType.DMA,
            pltpu.SemaphoreType.DMA,
        ],
        compiler_params=pltpu.CompilerParams(use_tc_tiling_on_sc=False),
    )(_emb_kernel)
    return kfn(x_flat, table)


def kernel(x, table):
    x_flat = x.astype(jnp.int32).reshape(NW, CHUNKS_PER_W, CHUNK)
    out = _embedding(x_flat, table)
    return out.reshape(x.shape[0], x.shape[1], D_MODEL)
